# Initial kernel scaffold; baseline (speedup 1.0000x reference)
import jax
import jax.numpy as jnp
from jax.experimental import pallas as pl


def kernel(X, edge_index, W_fst, W_snd, W_last, S_snd, S_last):
    raise NotImplementedError("write your pallas kernel here")


# speedup vs baseline: 4.8858x; 4.8858x over previous
import functools

import jax
import jax.numpy as jnp
from jax import lax
from jax.experimental import pallas as pl
from jax.experimental.pallas import tpu as pltpu
from jax.experimental.pallas import tpu_sc as plsc

N = 10000
E = 320000
DIN = 128
NCOUT = 121
NPAD = 10240
TRASH = N

NCORES = 2
NSUB = 16
NWORK = NCORES * NSUB
CHUNK = 128
EPW_CHUNKS = 80
EPW = CHUNK * EPW_CHUNKS
EPAD = NWORK * EPW
ROWS_PER_SUB = NPAD // NSUB
HW = 64
RBLK = 256
GRID = NPAD // RBLK

_f32 = jnp.float32


def _sc_mesh():
    return plsc.VectorSubcoreMesh(core_axis_name="c", subcore_axis_name="s",
                                  num_cores=NCORES, num_subcores=NSUB)


def _zero_fill(buf, nrows):
    ncol = buf.shape[1]

    def body(i, _):
        for j in range(ncol // 16):
            buf[i, pl.ds(j * 16, 16)] = jnp.zeros((16,), _f32)
        return 0

    lax.fori_loop(0, nrows, body, 0)


def _prop_body(tables, rowi, coli, out, rowv, colv, rows, zbuf, acc, sem):
    cid = lax.axis_index("c")
    sid = lax.axis_index("s")
    wid = sid * NCORES + cid

    _zero_fill(zbuf, 128)
    pltpu.sync_copy(rowi.at[pl.ds(wid * EPW_CHUNKS, EPW_CHUNKS)], rowv)
    pltpu.sync_copy(coli.at[pl.ds(wid * EPW_CHUNKS, EPW_CHUNKS)], colv)

    for h, table in enumerate(tables):
        for k in range(ROWS_PER_SUB // 128):
            pltpu.sync_copy(zbuf,
                            acc.at[pl.ds(sid * ROWS_PER_SUB + k * 128, 128)])
        plsc.subcore_barrier()

        def step(g, _):
            pltpu.async_copy(table.at[rowv.at[g]], rows, sem).wait()
            pltpu.sync_copy(rows, acc.at[colv.at[g]], add=True)
            return 0

        lax.fori_loop(0, EPW_CHUNKS, step, 0)
        plsc.subcore_barrier()

        for k in range(ROWS_PER_SUB // 128):
            r0 = sid * ROWS_PER_SUB + k * 128
            pltpu.sync_copy(acc.at[pl.ds(r0, 128)],
                            out.at[cid, h, pl.ds(r0, 128)])


@functools.cache
def _build_prop_sc(npass):
    def body(*refs):
        tables = refs[:npass]
        rowi, coli, out, rowv, colv, rows, zbuf, acc, sem = refs[npass:]
        _prop_body(tables, rowi, coli, out, rowv, colv, rows, zbuf, acc, sem)

    return pl.kernel(
        body,
        out_type=jax.ShapeDtypeStruct((NCORES, npass, NPAD, HW), _f32),
        mesh=_sc_mesh(),
        compiler_params=pltpu.CompilerParams(use_tc_tiling_on_sc=False),
        scratch_types=[
            pltpu.VMEM((EPW_CHUNKS, CHUNK), jnp.int32),
            pltpu.VMEM((EPW_CHUNKS, CHUNK), jnp.int32),
            pltpu.VMEM((CHUNK, HW), _f32),
            pltpu.VMEM((128, HW), _f32),
            pltpu.VMEM_SHARED((NPAD, HW), _f32),
            pltpu.SemaphoreType.DMA,
        ],
    )


def _prop_sc(tables, rowi, coli):
    return _build_prop_sc(len(tables))(*tables, rowi, coli)


def _deg_body(coli, out, colv, ones, zbuf, acc):
    cid = lax.axis_index("c")
    sid = lax.axis_index("s")
    wid = sid * NCORES + cid

    _zero_fill(zbuf, 128)

    def fill_ones(i, _):
        ones[i, pl.ds(0, 16)] = jnp.ones((16,), _f32)
        return 0

    lax.fori_loop(0, CHUNK, fill_ones, 0)

    for k in range(ROWS_PER_SUB // 128):
        pltpu.sync_copy(zbuf, acc.at[pl.ds(sid * ROWS_PER_SUB + k * 128, 128)])
    plsc.subcore_barrier()

    pltpu.sync_copy(coli.at[pl.ds(wid * EPW_CHUNKS, EPW_CHUNKS)], colv)

    def step(g, _):
        pltpu.sync_copy(ones, acc.at[colv.at[g]], add=True)
        return 0

    lax.fori_loop(0, EPW_CHUNKS, step, 0)
    plsc.subcore_barrier()

    for k in range(ROWS_PER_SUB // 128):
        r0 = sid * ROWS_PER_SUB + k * 128
        pltpu.sync_copy(acc.at[pl.ds(r0, 128)], out.at[cid, pl.ds(r0, 128)])


@functools.cache
def _build_deg_sc():
    return pl.kernel(
        _deg_body,
        out_type=jax.ShapeDtypeStruct((NCORES, NPAD, 16), _f32),
        mesh=_sc_mesh(),
        compiler_params=pltpu.CompilerParams(use_tc_tiling_on_sc=False),
        scratch_types=[
            pltpu.VMEM((EPW_CHUNKS, CHUNK), jnp.int32),
            pltpu.VMEM((CHUNK, 16), _f32),
            pltpu.VMEM((128, 16), _f32),
            pltpu.VMEM_SHARED((NPAD, 16), _f32),
        ],
    )


def _deg_sc(coli):
    return _build_deg_sc()(coli)


def _elu(x):
    return jnp.where(x > 0, x, jnp.exp(x) - 1.0)


def _psum(p_ref, dis, c):
    left = p_ref[0, 2 * c] + p_ref[1, 2 * c]
    right = p_ref[0, 2 * c + 1] + p_ref[1, 2 * c + 1]
    return dis * jnp.concatenate([left, right], axis=-1)


def _dis_kernel(degp_ref, x_ref, disb_ref, xs_ref):
    d = degp_ref[0] + degp_ref[1]
    dcol = d[:, 0:1]
    dis = jnp.where(dcol > 0, lax.rsqrt(dcol), 0.0)
    disb = jnp.broadcast_to(dis, (RBLK, 128))
    disb_ref[...] = disb
    x = disb * x_ref[...]
    xs_ref[0] = x[:, :HW]
    xs_ref[1] = x[:, HW:]


def _layer1_kernel(p1_ref, disb_ref, w1_ref, ofst_ref, f2c_ref):
    dis = disb_ref[...]
    p = _psum(p1_ref, dis, 0)
    h = jnp.dot(p, w1_ref[...], preferred_element_type=_f32)
    o = _elu(h)
    ofst_ref[...] = o
    for c in range(16):
        f2c_ref[c] = dis[:, :HW] * o[:, c * HW:(c + 1) * HW]


def _layer2_kernel(p2_ref, disb_ref, ofst_ref, w2_ref, s2_ref, wlm_ref,
                   osnd_ref, h3_ref):
    dis = disb_ref[...]
    acc = jnp.zeros((RBLK, 1024), _f32)
    for c in range(8):
        pc = _psum(p2_ref, dis, c)
        acc = acc + jnp.dot(pc, w2_ref[c], preferred_element_type=_f32)
        acc = acc + jnp.dot(ofst_ref[:, c * 128:(c + 1) * 128], s2_ref[c],
                            preferred_element_type=_f32)
    o = _elu(acc)
    osnd_ref[...] = o
    fo = jnp.concatenate([dis * o[:, c * 128:(c + 1) * 128] for c in range(8)],
                         axis=-1)
    h3 = jnp.dot(fo, wlm_ref[...], preferred_element_type=_f32)
    h3_ref[0] = h3[:, :HW]
    h3_ref[1] = h3[:, HW:]


def _layer3_kernel(p3_ref, disb_ref, osnd_ref, slm_ref, out_ref):
    dis = disb_ref[...]
    skip = jnp.dot(osnd_ref[...], slm_ref[...], preferred_element_type=_f32)
    out_ref[...] = _psum(p3_ref, dis, 0) + skip


def kernel(X, edge_index, W_fst, W_snd, W_last, S_snd, S_last):
    row = edge_index[0]
    col = edge_index[1]
    npad_e = EPAD - E
    row_p = jnp.concatenate([row, jnp.zeros((npad_e,), jnp.int32)])
    col_p = jnp.concatenate([col, jnp.full((npad_e,), TRASH, jnp.int32)])
    row2d = row_p.reshape(EPAD // CHUNK, CHUNK)
    col2d = col_p.reshape(EPAD // CHUNK, CHUNK)

    X_pad = jnp.pad(X, ((0, NPAD - N), (0, 0)))

    W1 = jnp.transpose(W_fst, (1, 0, 2)).reshape(DIN, 1024)
    W2 = jnp.transpose(W_snd, (1, 0, 2)).reshape(1024, 1024).reshape(8, 128, 1024)
    S2 = jnp.transpose(S_snd, (1, 0, 2)).reshape(1024, 1024).reshape(8, 128, 1024)
    Wlm = jnp.pad(jnp.mean(W_last, axis=0), ((0, 0), (0, 128 - NCOUT)))
    Slm = jnp.pad(jnp.mean(S_last, axis=0), ((0, 0), (0, 128 - NCOUT)))

    degp = _deg_sc(col2d)

    disb, xs2 = pl.pallas_call(
        _dis_kernel,
        grid=(GRID,),
        in_specs=[
            pl.BlockSpec((NCORES, RBLK, 16), lambda r: (0, r, 0)),
            pl.BlockSpec((RBLK, 128), lambda r: (r, 0)),
        ],
        out_specs=[
            pl.BlockSpec((RBLK, 128), lambda r: (r, 0)),
            pl.BlockSpec((2, RBLK, HW), lambda r: (0, r, 0)),
        ],
        out_shape=[
            jax.ShapeDtypeStruct((NPAD, 128), _f32),
            jax.ShapeDtypeStruct((2, NPAD, HW), _f32),
        ],
    )(degp, X_pad)

    p1 = _prop_sc([xs2[0], xs2[1]], row2d, col2d)

    out_fst, f2c = pl.pallas_call(
        _layer1_kernel,
        grid=(GRID,),
        in_specs=[
            pl.BlockSpec((NCORES, 2, RBLK, HW), lambda r: (0, 0, r, 0)),
            pl.BlockSpec((RBLK, 128), lambda r: (r, 0)),
            pl.BlockSpec((DIN, 1024), lambda r: (0, 0)),
        ],
        out_specs=[
            pl.BlockSpec((RBLK, 1024), lambda r: (r, 0)),
            pl.BlockSpec((16, RBLK, HW), lambda r: (0, r, 0)),
        ],
        out_shape=[
            jax.ShapeDtypeStruct((NPAD, 1024), _f32),
            jax.ShapeDtypeStruct((16, NPAD, HW), _f32),
        ],
    )(p1, disb, W1)

    p2 = _prop_sc([f2c[c] for c in range(16)], row2d, col2d)

    osnd, h3 = pl.pallas_call(
        _layer2_kernel,
        grid=(GRID,),
        in_specs=[
            pl.BlockSpec((NCORES, 16, RBLK, HW), lambda r: (0, 0, r, 0)),
            pl.BlockSpec((RBLK, 128), lambda r: (r, 0)),
            pl.BlockSpec((RBLK, 1024), lambda r: (r, 0)),
            pl.BlockSpec((8, 128, 1024), lambda r: (0, 0, 0)),
            pl.BlockSpec((8, 128, 1024), lambda r: (0, 0, 0)),
            pl.BlockSpec((1024, 128), lambda r: (0, 0)),
        ],
        out_specs=[
            pl.BlockSpec((RBLK, 1024), lambda r: (r, 0)),
            pl.BlockSpec((2, RBLK, HW), lambda r: (0, r, 0)),
        ],
        out_shape=[
            jax.ShapeDtypeStruct((NPAD, 1024), _f32),
            jax.ShapeDtypeStruct((2, NPAD, HW), _f32),
        ],
    )(p2, disb, out_fst, W2, S2, Wlm)

    p3 = _prop_sc([h3[0], h3[1]], row2d, col2d)

    out = pl.pallas_call(
        _layer3_kernel,
        grid=(GRID,),
        in_specs=[
            pl.BlockSpec((NCORES, 2, RBLK, HW), lambda r: (0, 0, r, 0)),
            pl.BlockSpec((RBLK, 128), lambda r: (r, 0)),
            pl.BlockSpec((RBLK, 1024), lambda r: (r, 0)),
            pl.BlockSpec((1024, 128), lambda r: (0, 0)),
        ],
        out_specs=pl.BlockSpec((RBLK, 128), lambda r: (r, 0)),
        out_shape=jax.ShapeDtypeStruct((NPAD, 128), _f32),
    )(p3, disb, osnd, Slm)

    return out[:N, :NCOUT]


# speedup vs baseline: 6.0225x; 1.2326x over previous
import functools

import jax
import jax.numpy as jnp
from jax import lax
from jax.experimental import pallas as pl
from jax.experimental.pallas import tpu as pltpu
from jax.experimental.pallas import tpu_sc as plsc

N = 10000
E = 320000
DIN = 128
NCOUT = 121
NPAD = 10240
TRASH = N

NCORES = 2
NSUB = 16
NWORK = NCORES * NSUB
CHUNK = 128
EPW_CHUNKS = 80
EPW = CHUNK * EPW_CHUNKS
EPAD = NWORK * EPW
ROWS_PER_SUB = NPAD // NSUB
HW = 64
RBLK = 256
GRID = NPAD // RBLK

_f32 = jnp.float32


def _sc_mesh():
    return plsc.VectorSubcoreMesh(core_axis_name="c", subcore_axis_name="s",
                                  num_cores=NCORES, num_subcores=NSUB)


def _zero_fill(buf, nrows):
    ncol = buf.shape[1]

    def body(i, _):
        for j in range(ncol // 16):
            buf[i, pl.ds(j * 16, 16)] = jnp.zeros((16,), _f32)
        return 0

    lax.fori_loop(0, nrows, body, 0)


NBUF = 8
RING_ROUNDS = EPW_CHUNKS // NBUF


def _prop_body(npass, tabs, rowi, coli, out, rowv, colv, rows, zbuf, acc,
               gsem, ssem):
    cid = lax.axis_index("c")
    sid = lax.axis_index("s")
    wid = sid * NCORES + cid

    _zero_fill(zbuf, 128)
    pltpu.sync_copy(rowi.at[pl.ds(wid * EPW_CHUNKS, EPW_CHUNKS)], rowv)
    pltpu.sync_copy(coli.at[pl.ds(wid * EPW_CHUNKS, EPW_CHUNKS)], colv)

    def one_pass(h, _):
        table = tabs.at[h]
        for k in range(ROWS_PER_SUB // 128):
            pltpu.async_copy(
                zbuf, acc.at[pl.ds(sid * ROWS_PER_SUB + k * 128, 128)],
                gsem.at[k])
        for k in range(ROWS_PER_SUB // 128):
            pltpu.make_async_copy(
                zbuf, acc.at[pl.ds(sid * ROWS_PER_SUB + k * 128, 128)],
                gsem.at[k]).wait()
        plsc.subcore_barrier()

        for b in range(NBUF):
            pltpu.async_copy(table.at[rowv.at[b]], rows.at[b], gsem.at[b])

        def ring(i, _):
            for b in range(NBUF):
                g = i * NBUF + b
                pltpu.make_async_copy(table.at[rowv.at[g]], rows.at[b],
                                      gsem.at[b]).wait()
                pltpu.async_copy(rows.at[b], acc.at[colv.at[g]], ssem.at[b],
                                 add=True)
                pltpu.make_async_copy(rows.at[b], acc.at[colv.at[g]],
                                      ssem.at[b]).wait()
                pltpu.async_copy(table.at[rowv.at[g + NBUF]], rows.at[b],
                                 gsem.at[b])
            return 0

        lax.fori_loop(0, RING_ROUNDS - 1, ring, 0)
        for b in range(NBUF):
            g = (RING_ROUNDS - 1) * NBUF + b
            pltpu.make_async_copy(table.at[rowv.at[g]], rows.at[b],
                                  gsem.at[b]).wait()
            pltpu.async_copy(rows.at[b], acc.at[colv.at[g]], ssem.at[b],
                             add=True)
            pltpu.make_async_copy(rows.at[b], acc.at[colv.at[g]],
                                  ssem.at[b]).wait()
        plsc.subcore_barrier()

        for k in range(ROWS_PER_SUB // 128):
            r0 = sid * ROWS_PER_SUB + k * 128
            pltpu.async_copy(acc.at[pl.ds(r0, 128)],
                             out.at[cid, h, pl.ds(r0, 128)], ssem.at[k])
        for k in range(ROWS_PER_SUB // 128):
            r0 = sid * ROWS_PER_SUB + k * 128
            pltpu.make_async_copy(acc.at[pl.ds(r0, 128)],
                                  out.at[cid, h, pl.ds(r0, 128)],
                                  ssem.at[k]).wait()
        return 0

    lax.fori_loop(0, npass, one_pass, 0)


@functools.cache
def _build_prop_sc(npass, hw):
    def body(tabs, rowi, coli, out, rowv, colv, rows, zbuf, acc, gsem, ssem):
        _prop_body(npass, tabs, rowi, coli, out, rowv, colv, rows, zbuf, acc,
                   gsem, ssem)

    return pl.kernel(
        body,
        out_type=jax.ShapeDtypeStruct((NCORES, npass, NPAD, hw), _f32),
        mesh=_sc_mesh(),
        compiler_params=pltpu.CompilerParams(use_tc_tiling_on_sc=False),
        scratch_types=[
            pltpu.VMEM((EPW_CHUNKS, CHUNK), jnp.int32),
            pltpu.VMEM((EPW_CHUNKS, CHUNK), jnp.int32),
            pltpu.VMEM((NBUF, CHUNK, hw), _f32),
            pltpu.VMEM((128, hw), _f32),
            pltpu.VMEM_SHARED((NPAD, hw), _f32),
            pltpu.SemaphoreType.DMA((NBUF,)),
            pltpu.SemaphoreType.DMA((NBUF,)),
        ],
    )


def _prop_sc(tabs, rowi, coli):
    npass, _, hw = tabs.shape
    return _build_prop_sc(npass, hw)(tabs, rowi, coli)


def _deg_body(coli, out, colv, ones, zbuf, acc, dsem):
    cid = lax.axis_index("c")
    sid = lax.axis_index("s")
    wid = sid * NCORES + cid

    _zero_fill(zbuf, 128)

    def fill_ones(i, _):
        ones[i, pl.ds(0, 16)] = jnp.ones((16,), _f32)
        return 0

    lax.fori_loop(0, CHUNK, fill_ones, 0)

    for k in range(ROWS_PER_SUB // 128):
        pltpu.sync_copy(zbuf, acc.at[pl.ds(sid * ROWS_PER_SUB + k * 128, 128)])
    plsc.subcore_barrier()

    pltpu.sync_copy(coli.at[pl.ds(wid * EPW_CHUNKS, EPW_CHUNKS)], colv)

    def step(i, _):
        for b in range(NBUF):
            pltpu.async_copy(ones, acc.at[colv.at[i * NBUF + b]], dsem,
                             add=True)
        for b in range(NBUF):
            pltpu.make_async_copy(ones, acc.at[colv.at[i * NBUF + b]],
                                  dsem).wait()
        return 0

    lax.fori_loop(0, EPW_CHUNKS // NBUF, step, 0)
    plsc.subcore_barrier()

    for k in range(ROWS_PER_SUB // 128):
        r0 = sid * ROWS_PER_SUB + k * 128
        pltpu.sync_copy(acc.at[pl.ds(r0, 128)], out.at[cid, pl.ds(r0, 128)])


@functools.cache
def _build_deg_sc():
    return pl.kernel(
        _deg_body,
        out_type=jax.ShapeDtypeStruct((NCORES, NPAD, 16), _f32),
        mesh=_sc_mesh(),
        compiler_params=pltpu.CompilerParams(use_tc_tiling_on_sc=False),
        scratch_types=[
            pltpu.VMEM((EPW_CHUNKS, CHUNK), jnp.int32),
            pltpu.VMEM((CHUNK, 16), _f32),
            pltpu.VMEM((128, 16), _f32),
            pltpu.VMEM_SHARED((NPAD, 16), _f32),
            pltpu.SemaphoreType.DMA,
        ],
    )


def _deg_sc(coli):
    return _build_deg_sc()(coli)


def _elu(x):
    return jnp.where(x > 0, x, jnp.exp(x) - 1.0)


def _pchunk(p_ref, dis, c, w):
    k = 128 // w
    parts = [p_ref[0, c * k + j] + p_ref[1, c * k + j] for j in range(k)]
    return dis * jnp.concatenate(parts, axis=-1)


def _dis_kernel(degp_ref, x_ref, disb_ref, xs_ref):
    d = degp_ref[0] + degp_ref[1]
    dcol = d[:, 0:1]
    dis = jnp.where(dcol > 0, lax.rsqrt(dcol), 0.0)
    disb = jnp.broadcast_to(dis, (RBLK, 128))
    disb_ref[...] = disb
    x = disb * x_ref[...]
    for j in range(4):
        xs_ref[j] = x[:, j * 32:(j + 1) * 32]


def _layer1_kernel(p1_ref, disb_ref, w1_ref, ofst_ref, f2c_ref):
    dis = disb_ref[...]
    p = _pchunk(p1_ref, dis, 0, 32)
    h = jnp.dot(p, w1_ref[...], preferred_element_type=_f32)
    o = _elu(h)
    ofst_ref[...] = o
    for c in range(32):
        f2c_ref[c] = dis[:, :32] * o[:, c * 32:(c + 1) * 32]


def _layer2_kernel(p2_ref, disb_ref, ofst_ref, w2_ref, s2_ref, wlm_ref,
                   osnd_ref, h3_ref):
    dis = disb_ref[...]
    acc = jnp.zeros((RBLK, 1024), _f32)
    for c in range(8):
        pc = _pchunk(p2_ref, dis, c, 32)
        acc = acc + jnp.dot(pc, w2_ref[c], preferred_element_type=_f32)
        acc = acc + jnp.dot(ofst_ref[:, c * 128:(c + 1) * 128], s2_ref[c],
                            preferred_element_type=_f32)
    o = _elu(acc)
    osnd_ref[...] = o
    fo = jnp.concatenate([dis * o[:, c * 128:(c + 1) * 128] for c in range(8)],
                         axis=-1)
    h3 = jnp.dot(fo, wlm_ref[...], preferred_element_type=_f32)
    for j in range(4):
        h3_ref[j] = h3[:, j * 32:(j + 1) * 32]


def _layer3_kernel(p3_ref, disb_ref, osnd_ref, slm_ref, out_ref):
    dis = disb_ref[...]
    skip = jnp.dot(osnd_ref[...], slm_ref[...], preferred_element_type=_f32)
    out_ref[...] = _pchunk(p3_ref, dis, 0, 32) + skip


def kernel(X, edge_index, W_fst, W_snd, W_last, S_snd, S_last):
    row = edge_index[0]
    col = edge_index[1]
    npad_e = EPAD - E
    row_p = jnp.concatenate([row, jnp.zeros((npad_e,), jnp.int32)])
    col_p = jnp.concatenate([col, jnp.full((npad_e,), TRASH, jnp.int32)])
    row2d = row_p.reshape(EPAD // CHUNK, CHUNK)
    col2d = col_p.reshape(EPAD // CHUNK, CHUNK)

    X_pad = jnp.pad(X, ((0, NPAD - N), (0, 0)))

    W1 = jnp.transpose(W_fst, (1, 0, 2)).reshape(DIN, 1024)
    W2 = jnp.transpose(W_snd, (1, 0, 2)).reshape(1024, 1024).reshape(8, 128, 1024)
    S2 = jnp.transpose(S_snd, (1, 0, 2)).reshape(1024, 1024).reshape(8, 128, 1024)
    Wlm = jnp.pad(jnp.mean(W_last, axis=0), ((0, 0), (0, 128 - NCOUT)))
    Slm = jnp.pad(jnp.mean(S_last, axis=0), ((0, 0), (0, 128 - NCOUT)))

    degp = _deg_sc(col2d)

    disb, xs2 = pl.pallas_call(
        _dis_kernel,
        grid=(GRID,),
        in_specs=[
            pl.BlockSpec((NCORES, RBLK, 16), lambda r: (0, r, 0)),
            pl.BlockSpec((RBLK, 128), lambda r: (r, 0)),
        ],
        out_specs=[
            pl.BlockSpec((RBLK, 128), lambda r: (r, 0)),
            pl.BlockSpec((4, RBLK, 32), lambda r: (0, r, 0)),
        ],
        out_shape=[
            jax.ShapeDtypeStruct((NPAD, 128), _f32),
            jax.ShapeDtypeStruct((4, NPAD, 32), _f32),
        ],
    )(degp, X_pad)

    p1 = _prop_sc(xs2, row2d, col2d)

    out_fst, f2c = pl.pallas_call(
        _layer1_kernel,
        grid=(GRID,),
        in_specs=[
            pl.BlockSpec((NCORES, 4, RBLK, 32), lambda r: (0, 0, r, 0)),
            pl.BlockSpec((RBLK, 128), lambda r: (r, 0)),
            pl.BlockSpec((DIN, 1024), lambda r: (0, 0)),
        ],
        out_specs=[
            pl.BlockSpec((RBLK, 1024), lambda r: (r, 0)),
            pl.BlockSpec((32, RBLK, 32), lambda r: (0, r, 0)),
        ],
        out_shape=[
            jax.ShapeDtypeStruct((NPAD, 1024), _f32),
            jax.ShapeDtypeStruct((32, NPAD, 32), _f32),
        ],
    )(p1, disb, W1)

    p2 = _prop_sc(f2c, row2d, col2d)

    osnd, h3 = pl.pallas_call(
        _layer2_kernel,
        grid=(GRID,),
        in_specs=[
            pl.BlockSpec((NCORES, 32, RBLK, 32), lambda r: (0, 0, r, 0)),
            pl.BlockSpec((RBLK, 128), lambda r: (r, 0)),
            pl.BlockSpec((RBLK, 1024), lambda r: (r, 0)),
            pl.BlockSpec((8, 128, 1024), lambda r: (0, 0, 0)),
            pl.BlockSpec((8, 128, 1024), lambda r: (0, 0, 0)),
            pl.BlockSpec((1024, 128), lambda r: (0, 0)),
        ],
        out_specs=[
            pl.BlockSpec((RBLK, 1024), lambda r: (r, 0)),
            pl.BlockSpec((4, RBLK, 32), lambda r: (0, r, 0)),
        ],
        out_shape=[
            jax.ShapeDtypeStruct((NPAD, 1024), _f32),
            jax.ShapeDtypeStruct((4, NPAD, 32), _f32),
        ],
    )(p2, disb, out_fst, W2, S2, Wlm)

    p3 = _prop_sc(h3, row2d, col2d)

    out = pl.pallas_call(
        _layer3_kernel,
        grid=(GRID,),
        in_specs=[
            pl.BlockSpec((NCORES, 4, RBLK, 32), lambda r: (0, 0, r, 0)),
            pl.BlockSpec((RBLK, 128), lambda r: (r, 0)),
            pl.BlockSpec((RBLK, 1024), lambda r: (r, 0)),
            pl.BlockSpec((1024, 128), lambda r: (0, 0)),
        ],
        out_specs=pl.BlockSpec((RBLK, 128), lambda r: (r, 0)),
        out_shape=jax.ShapeDtypeStruct((NPAD, 128), _f32),
    )(p3, disb, osnd, Slm)

    return out[:N, :NCOUT]
